# baseline (speedup 1.0000x reference)
import jax
import jax.numpy as jnp
from jax import lax
from jax.experimental import pallas as pl
from jax.experimental.pallas import tpu as pltpu
from jax.experimental.pallas import tpu_sc as plsc

N_TOKENS = 32768
D_MODEL = 768
ENC_DIM = 192
N_EXPERTS = 8
LN_EPS = 1e-5

BLK = 2048

_NC = 2
_NS = 16
_NW = _NC * _NS
_TPW = N_TOKENS // _NW
_GRP = _TPW // 16


def _tc_body(hsa_ref, hsb_ref, hsc_ref, hsd_ref, wproj_ref, bproj_ref,
             gamma_ref, beta_ref, wg_ref, out_t_ref, tbl_ref):
    wg = wg_ref[...]
    wgh = wg[:D_MODEL, :]
    wge = wg[D_MODEL:, :]

    a = wproj_ref[...]
    b = bproj_ref[...]
    g = gamma_ref[...]
    bet = beta_ref[...]
    sel = (lax.broadcasted_iota(jnp.int32, (8, 1), 0) == 1)
    enc = jnp.where(sel, a + b, b * jnp.ones((8, 1), jnp.float32))
    mu = jnp.mean(enc, axis=-1, keepdims=True)
    var = jnp.mean((enc - mu) ** 2, axis=-1, keepdims=True)
    enc = (enc - mu) / jnp.sqrt(var + LN_EPS) * g + bet
    tbl8 = jnp.dot(enc, wge, preferred_element_type=jnp.float32)
    tbl_ref[...] = jnp.concatenate([tbl8[0:1, :], tbl8[1:2, :]], axis=1)

    for k, ref in enumerate((hsa_ref, hsb_ref, hsc_ref, hsd_ref)):
        lk = jnp.dot(ref[...], wgh, preferred_element_type=jnp.float32)
        out_t_ref[:, k * BLK:(k + 1) * BLK] = lk.T


def _tc_logits_t(hs, w_proj, b_proj, gamma, beta, w_gate):
    grid = N_TOKENS // (4 * BLK)
    return pl.pallas_call(
        _tc_body,
        grid=(grid,),
        in_specs=[
            pl.BlockSpec((BLK, D_MODEL), lambda i: (4 * i, 0)),
            pl.BlockSpec((BLK, D_MODEL), lambda i: (4 * i + 1, 0)),
            pl.BlockSpec((BLK, D_MODEL), lambda i: (4 * i + 2, 0)),
            pl.BlockSpec((BLK, D_MODEL), lambda i: (4 * i + 3, 0)),
            pl.BlockSpec((1, ENC_DIM), lambda i: (0, 0)),
            pl.BlockSpec((1, ENC_DIM), lambda i: (0, 0)),
            pl.BlockSpec((1, ENC_DIM), lambda i: (0, 0)),
            pl.BlockSpec((1, ENC_DIM), lambda i: (0, 0)),
            pl.BlockSpec((D_MODEL + ENC_DIM, N_EXPERTS), lambda i: (0, 0)),
        ],
        out_specs=[
            pl.BlockSpec((N_EXPERTS, 4 * BLK), lambda i: (0, i)),
            pl.BlockSpec((1, 16), lambda i: (0, 0)),
        ],
        out_shape=[
            jax.ShapeDtypeStruct((N_EXPERTS, N_TOKENS), jnp.float32),
            jax.ShapeDtypeStruct((1, 16), jnp.float32),
        ],
        compiler_params=pltpu.CompilerParams(
            dimension_semantics=("arbitrary",),
        ),
    )(hs, hs, hs, hs, w_proj, b_proj, gamma, beta, w_gate)


def _sc_topk_body(lt_hbm, ea_hbm, tbl_hbm, lo_hbm, i2_hbm, w2_hbm,
                  col_v, ea_v, tbl_v, lo_v, i2_v, w2_v, sem):
    wid = lax.axis_index("s") * _NC + lax.axis_index("c")
    base = wid * _TPW
    cin = pltpu.async_copy(lt_hbm.at[:, pl.ds(base, _TPW)], col_v, sem)
    cea = pltpu.async_copy(ea_hbm.at[pl.ds(base, _TPW)], ea_v, sem)
    ctb = pltpu.async_copy(tbl_hbm, tbl_v, sem)
    cin.wait()
    cea.wait()
    ctb.wait()

    tv = tbl_v[0, pl.ds(0, 16)]
    t0 = [tv[e] for e in range(N_EXPERTS)]
    dt = [tv[8 + e] - tv[e] for e in range(N_EXPERTS)]

    def group(g, _):
        off = g * 16
        ea = ea_v[pl.ds(off, 16)]
        cols = []
        for e in range(N_EXPERTS):
            c = col_v[e, pl.ds(off, 16)] + (jnp.full((16,), t0[e], jnp.float32)
                                            + ea * dt[e])
            lo_v[e, pl.ds(off, 16)] = c
            cols.append(c)
        m1 = cols[0]
        i1 = jnp.zeros((16,), jnp.int32)
        m2 = jnp.full((16,), -jnp.inf, jnp.float32)
        i2 = jnp.zeros((16,), jnp.int32)
        for e in range(1, N_EXPERTS):
            v = cols[e]
            ev = jnp.full((16,), e, jnp.int32)
            gt1 = v > m1
            gt2 = v > m2
            m2n = jnp.where(gt1, m1, jnp.where(gt2, v, m2))
            i2n = jnp.where(gt1, i1, jnp.where(gt2, ev, i2))
            m1 = jnp.where(gt1, v, m1)
            i1 = jnp.where(gt1, ev, i1)
            m2, i2 = m2n, i2n
        t = jnp.exp(m2 - m1)
        s = 1.0 + t
        i2_v[0, pl.ds(off, 16)] = i1
        i2_v[1, pl.ds(off, 16)] = i2
        w2_v[0, pl.ds(off, 16)] = 1.0 / s
        w2_v[1, pl.ds(off, 16)] = t / s
        return 0

    lax.fori_loop(0, _GRP, group, 0)

    c1 = pltpu.async_copy(lo_v, lo_hbm.at[:, pl.ds(base, _TPW)], sem)
    c2 = pltpu.async_copy(i2_v, i2_hbm.at[:, pl.ds(base, _TPW)], sem)
    c3 = pltpu.async_copy(w2_v, w2_hbm.at[:, pl.ds(base, _TPW)], sem)
    c1.wait()
    c2.wait()
    c3.wait()


def _sc_topk(logits_t, ea_flat, tbl):
    mesh = plsc.VectorSubcoreMesh(core_axis_name="c", subcore_axis_name="s")
    f = pl.kernel(
        _sc_topk_body,
        mesh=mesh,
        out_type=[
            jax.ShapeDtypeStruct((N_EXPERTS, N_TOKENS), jnp.float32),
            jax.ShapeDtypeStruct((2, N_TOKENS), jnp.int32),
            jax.ShapeDtypeStruct((2, N_TOKENS), jnp.float32),
        ],
        scratch_types=[
            pltpu.VMEM((N_EXPERTS, _TPW), jnp.float32),
            pltpu.VMEM((_TPW,), jnp.float32),
            pltpu.VMEM((1, 16), jnp.float32),
            pltpu.VMEM((N_EXPERTS, _TPW), jnp.float32),
            pltpu.VMEM((2, _TPW), jnp.int32),
            pltpu.VMEM((2, _TPW), jnp.float32),
            pltpu.SemaphoreType.DMA,
        ],
    )
    return f(logits_t, ea_flat, tbl)


def kernel(hidden_states, encoder_available, W_proj, b_proj, gamma, beta, W_gate):
    hs = hidden_states.astype(jnp.float32)
    ea_flat = encoder_available.astype(jnp.float32).reshape(N_TOKENS)
    lt_h, tbl = _tc_logits_t(hs,
                             W_proj.reshape(1, ENC_DIM),
                             b_proj.reshape(1, ENC_DIM),
                             gamma.reshape(1, ENC_DIM),
                             beta.reshape(1, ENC_DIM),
                             W_gate)
    lt, idx2, w2 = _sc_topk(lt_h, ea_flat, tbl)
    return (idx2.T, w2.T, lt.T)


# speedup vs baseline: 1.0067x; 1.0067x over previous
import jax
import jax.numpy as jnp
from jax import lax
from jax.experimental import pallas as pl
from jax.experimental.pallas import tpu as pltpu
from jax.experimental.pallas import tpu_sc as plsc

N_TOKENS = 32768
D_MODEL = 768
ENC_DIM = 192
N_EXPERTS = 8
LN_EPS = 1e-5

BLK = 1024

_NC = 2
_NS = 16
_NW = _NC * _NS
_TPW = N_TOKENS // _NW
_GRP = _TPW // 16


def _tc_body(hsa_ref, hsb_ref, wproj_ref, bproj_ref,
             gamma_ref, beta_ref, wg_ref, out_t_ref, tbl_ref):
    wg = wg_ref[...]
    wgh = wg[:D_MODEL, :]
    wge = wg[D_MODEL:, :]

    a = wproj_ref[...]
    b = bproj_ref[...]
    g = gamma_ref[...]
    bet = beta_ref[...]
    sel = (lax.broadcasted_iota(jnp.int32, (8, 1), 0) == 1)
    enc = jnp.where(sel, a + b, b * jnp.ones((8, 1), jnp.float32))
    mu = jnp.mean(enc, axis=-1, keepdims=True)
    var = jnp.mean((enc - mu) ** 2, axis=-1, keepdims=True)
    enc = (enc - mu) / jnp.sqrt(var + LN_EPS) * g + bet
    tbl8 = jnp.dot(enc, wge, preferred_element_type=jnp.float32)
    tbl_ref[...] = jnp.concatenate([tbl8[0:1, :], tbl8[1:2, :]], axis=1)

    for k, ref in enumerate((hsa_ref, hsb_ref)):
        lk = jnp.dot(ref[...], wgh, preferred_element_type=jnp.float32)
        out_t_ref[:, k * BLK:(k + 1) * BLK] = lk.T


def _tc_logits_t(hs, w_proj, b_proj, gamma, beta, w_gate):
    grid = N_TOKENS // (2 * BLK)
    return pl.pallas_call(
        _tc_body,
        grid=(grid,),
        in_specs=[
            pl.BlockSpec((BLK, D_MODEL), lambda i: (2 * i, 0)),
            pl.BlockSpec((BLK, D_MODEL), lambda i: (2 * i + 1, 0)),
            pl.BlockSpec((1, ENC_DIM), lambda i: (0, 0)),
            pl.BlockSpec((1, ENC_DIM), lambda i: (0, 0)),
            pl.BlockSpec((1, ENC_DIM), lambda i: (0, 0)),
            pl.BlockSpec((1, ENC_DIM), lambda i: (0, 0)),
            pl.BlockSpec((D_MODEL + ENC_DIM, N_EXPERTS), lambda i: (0, 0)),
        ],
        out_specs=[
            pl.BlockSpec((N_EXPERTS, 2 * BLK), lambda i: (0, i)),
            pl.BlockSpec((1, 16), lambda i: (0, 0)),
        ],
        out_shape=[
            jax.ShapeDtypeStruct((N_EXPERTS, N_TOKENS), jnp.float32),
            jax.ShapeDtypeStruct((1, 16), jnp.float32),
        ],
        compiler_params=pltpu.CompilerParams(
            dimension_semantics=("arbitrary",),
        ),
    )(hs, hs, w_proj, b_proj, gamma, beta, w_gate)


def _sc_topk_body(lt_hbm, ea_hbm, tbl_hbm, lo_hbm, i2_hbm, w2_hbm,
                  col_v, ea_v, tbl_v, lo_v, i2_v, w2_v, sem):
    wid = lax.axis_index("s") * _NC + lax.axis_index("c")
    base = wid * _TPW
    cin = pltpu.async_copy(lt_hbm.at[:, pl.ds(base, _TPW)], col_v, sem)
    cea = pltpu.async_copy(ea_hbm.at[pl.ds(base, _TPW)], ea_v, sem)
    ctb = pltpu.async_copy(tbl_hbm, tbl_v, sem)
    cin.wait()
    cea.wait()
    ctb.wait()

    tv = tbl_v[0, pl.ds(0, 16)]
    t0 = [tv[e] for e in range(N_EXPERTS)]
    dt = [tv[8 + e] - tv[e] for e in range(N_EXPERTS)]

    def group(g, _):
        off = g * 16
        ea = ea_v[pl.ds(off, 16)]
        cols = []
        for e in range(N_EXPERTS):
            c = col_v[e, pl.ds(off, 16)] + (jnp.full((16,), t0[e], jnp.float32)
                                            + ea * dt[e])
            lo_v[e, pl.ds(off, 16)] = c
            cols.append(c)
        m1 = cols[0]
        i1 = jnp.zeros((16,), jnp.int32)
        m2 = jnp.full((16,), -jnp.inf, jnp.float32)
        i2 = jnp.zeros((16,), jnp.int32)
        for e in range(1, N_EXPERTS):
            v = cols[e]
            ev = jnp.full((16,), e, jnp.int32)
            gt1 = v > m1
            gt2 = v > m2
            m2n = jnp.where(gt1, m1, jnp.where(gt2, v, m2))
            i2n = jnp.where(gt1, i1, jnp.where(gt2, ev, i2))
            m1 = jnp.where(gt1, v, m1)
            i1 = jnp.where(gt1, ev, i1)
            m2, i2 = m2n, i2n
        t = jnp.exp(m2 - m1)
        s = 1.0 + t
        i2_v[0, pl.ds(off, 16)] = i1
        i2_v[1, pl.ds(off, 16)] = i2
        w2_v[0, pl.ds(off, 16)] = 1.0 / s
        w2_v[1, pl.ds(off, 16)] = t / s
        return 0

    lax.fori_loop(0, _GRP, group, 0)

    c1 = pltpu.async_copy(lo_v, lo_hbm.at[:, pl.ds(base, _TPW)], sem)
    c2 = pltpu.async_copy(i2_v, i2_hbm.at[:, pl.ds(base, _TPW)], sem)
    c3 = pltpu.async_copy(w2_v, w2_hbm.at[:, pl.ds(base, _TPW)], sem)
    c1.wait()
    c2.wait()
    c3.wait()


def _sc_topk(logits_t, ea_flat, tbl):
    mesh = plsc.VectorSubcoreMesh(core_axis_name="c", subcore_axis_name="s")
    f = pl.kernel(
        _sc_topk_body,
        mesh=mesh,
        out_type=[
            jax.ShapeDtypeStruct((N_EXPERTS, N_TOKENS), jnp.float32),
            jax.ShapeDtypeStruct((2, N_TOKENS), jnp.int32),
            jax.ShapeDtypeStruct((2, N_TOKENS), jnp.float32),
        ],
        scratch_types=[
            pltpu.VMEM((N_EXPERTS, _TPW), jnp.float32),
            pltpu.VMEM((_TPW,), jnp.float32),
            pltpu.VMEM((1, 16), jnp.float32),
            pltpu.VMEM((N_EXPERTS, _TPW), jnp.float32),
            pltpu.VMEM((2, _TPW), jnp.int32),
            pltpu.VMEM((2, _TPW), jnp.float32),
            pltpu.SemaphoreType.DMA,
        ],
    )
    return f(logits_t, ea_flat, tbl)


def kernel(hidden_states, encoder_available, W_proj, b_proj, gamma, beta, W_gate):
    hs = hidden_states.astype(jnp.float32)
    ea_flat = encoder_available.astype(jnp.float32).reshape(N_TOKENS)
    lt_h, tbl = _tc_logits_t(hs,
                             W_proj.reshape(1, ENC_DIM),
                             b_proj.reshape(1, ENC_DIM),
                             gamma.reshape(1, ENC_DIM),
                             beta.reshape(1, ENC_DIM),
                             W_gate)
    lt, idx2, w2 = _sc_topk(lt_h, ea_flat, tbl)
    return (idx2.T, w2.T, lt.T)


# speedup vs baseline: 1.0741x; 1.0669x over previous
import jax
import jax.numpy as jnp
from jax import lax
from jax.experimental import pallas as pl
from jax.experimental.pallas import tpu as pltpu
from jax.experimental.pallas import tpu_sc as plsc

N_TOKENS = 32768
D_MODEL = 768
ENC_DIM = 192
N_EXPERTS = 8
LN_EPS = 1e-5

BLK = 2048

_NC = 2
_NS = 16
_NW = _NC * _NS
_TPW = N_TOKENS // _NW
_GRP = _TPW // 16


def _tc_body(hsa_ref, hsb_ref, wproj_ref, bproj_ref,
             gamma_ref, beta_ref, wgt_ref, out_t_ref, tbl_ref):
    wg = wgt_ref[...].T
    wgh = wg[:D_MODEL, :]
    wge = wg[D_MODEL:, :]

    a = wproj_ref[...]
    b = bproj_ref[...]
    g = gamma_ref[...]
    bet = beta_ref[...]
    sel = (lax.broadcasted_iota(jnp.int32, (8, 1), 0) == 1)
    enc = jnp.where(sel, a + b, b * jnp.ones((8, 1), jnp.float32))
    mu = jnp.mean(enc, axis=-1, keepdims=True)
    var = jnp.mean((enc - mu) ** 2, axis=-1, keepdims=True)
    enc = (enc - mu) / jnp.sqrt(var + LN_EPS) * g + bet
    tbl8 = jnp.dot(enc, wge, preferred_element_type=jnp.float32)
    tbl_ref[...] = jnp.concatenate([tbl8[0:1, :], tbl8[1:2, :]], axis=1)

    for k, ref in enumerate((hsa_ref, hsb_ref)):
        lk = jnp.dot(ref[...], wgh, preferred_element_type=jnp.float32)
        out_t_ref[:, k * BLK:(k + 1) * BLK] = lk.T


def _tc_logits_t(hs, w_proj, b_proj, gamma, beta, w_gate):
    grid = N_TOKENS // (2 * BLK)
    return pl.pallas_call(
        _tc_body,
        grid=(grid,),
        in_specs=[
            pl.BlockSpec((BLK, D_MODEL), lambda i: (2 * i, 0)),
            pl.BlockSpec((BLK, D_MODEL), lambda i: (2 * i + 1, 0)),
            pl.BlockSpec((1, ENC_DIM), lambda i: (0, 0)),
            pl.BlockSpec((1, ENC_DIM), lambda i: (0, 0)),
            pl.BlockSpec((1, ENC_DIM), lambda i: (0, 0)),
            pl.BlockSpec((1, ENC_DIM), lambda i: (0, 0)),
            pl.BlockSpec((N_EXPERTS, D_MODEL + ENC_DIM), lambda i: (0, 0)),
        ],
        out_specs=[
            pl.BlockSpec((N_EXPERTS, 2 * BLK), lambda i: (0, i)),
            pl.BlockSpec((1, 16), lambda i: (0, 0)),
        ],
        out_shape=[
            jax.ShapeDtypeStruct((N_EXPERTS, N_TOKENS), jnp.float32),
            jax.ShapeDtypeStruct((1, 16), jnp.float32),
        ],
        compiler_params=pltpu.CompilerParams(
            dimension_semantics=("arbitrary",),
        ),
    )(hs, hs, w_proj, b_proj, gamma, beta, w_gate.T)


def _sc_topk_body(lt_hbm, ea_hbm, tbl_hbm, lo_hbm, i2_hbm, w2_hbm,
                  col_v, ea_v, tbl_v, lo_v, i2_v, w2_v, sem):
    wid = lax.axis_index("s") * _NC + lax.axis_index("c")
    base = wid * _TPW
    cin = pltpu.async_copy(lt_hbm.at[:, pl.ds(base, _TPW)], col_v, sem)
    cea = pltpu.async_copy(ea_hbm.at[pl.ds(base, _TPW)], ea_v, sem)
    ctb = pltpu.async_copy(tbl_hbm, tbl_v, sem)
    cin.wait()
    cea.wait()
    ctb.wait()

    tv = tbl_v[0, pl.ds(0, 16)]
    t0 = [tv[e] for e in range(N_EXPERTS)]
    dt = [tv[8 + e] - tv[e] for e in range(N_EXPERTS)]

    def group(g, _):
        off = g * 16
        ea = ea_v[pl.ds(off, 16)]
        cols = []
        for e in range(N_EXPERTS):
            c = col_v[e, pl.ds(off, 16)] + (jnp.full((16,), t0[e], jnp.float32)
                                            + ea * dt[e])
            lo_v[e, pl.ds(off, 16)] = c
            cols.append(c)
        m1 = cols[0]
        i1 = jnp.zeros((16,), jnp.int32)
        m2 = jnp.full((16,), -jnp.inf, jnp.float32)
        i2 = jnp.zeros((16,), jnp.int32)
        for e in range(1, N_EXPERTS):
            v = cols[e]
            ev = jnp.full((16,), e, jnp.int32)
            gt1 = v > m1
            gt2 = v > m2
            m2n = jnp.where(gt1, m1, jnp.where(gt2, v, m2))
            i2n = jnp.where(gt1, i1, jnp.where(gt2, ev, i2))
            m1 = jnp.where(gt1, v, m1)
            i1 = jnp.where(gt1, ev, i1)
            m2, i2 = m2n, i2n
        t = jnp.exp(m2 - m1)
        s = 1.0 + t
        i2_v[0, pl.ds(off, 16)] = i1
        i2_v[1, pl.ds(off, 16)] = i2
        w2_v[0, pl.ds(off, 16)] = 1.0 / s
        w2_v[1, pl.ds(off, 16)] = t / s
        return 0

    lax.fori_loop(0, _GRP, group, 0)

    c1 = pltpu.async_copy(lo_v, lo_hbm.at[:, pl.ds(base, _TPW)], sem)
    c2 = pltpu.async_copy(i2_v, i2_hbm.at[:, pl.ds(base, _TPW)], sem)
    c3 = pltpu.async_copy(w2_v, w2_hbm.at[:, pl.ds(base, _TPW)], sem)
    c1.wait()
    c2.wait()
    c3.wait()


def _sc_topk(logits_t, ea_flat, tbl):
    mesh = plsc.VectorSubcoreMesh(core_axis_name="c", subcore_axis_name="s")
    f = pl.kernel(
        _sc_topk_body,
        mesh=mesh,
        out_type=[
            jax.ShapeDtypeStruct((N_EXPERTS, N_TOKENS), jnp.float32),
            jax.ShapeDtypeStruct((2, N_TOKENS), jnp.int32),
            jax.ShapeDtypeStruct((2, N_TOKENS), jnp.float32),
        ],
        scratch_types=[
            pltpu.VMEM((N_EXPERTS, _TPW), jnp.float32),
            pltpu.VMEM((_TPW,), jnp.float32),
            pltpu.VMEM((1, 16), jnp.float32),
            pltpu.VMEM((N_EXPERTS, _TPW), jnp.float32),
            pltpu.VMEM((2, _TPW), jnp.int32),
            pltpu.VMEM((2, _TPW), jnp.float32),
            pltpu.SemaphoreType.DMA,
        ],
    )
    return f(logits_t, ea_flat, tbl)


def kernel(hidden_states, encoder_available, W_proj, b_proj, gamma, beta, W_gate):
    hs = hidden_states.astype(jnp.float32)
    ea_flat = encoder_available.astype(jnp.float32).reshape(N_TOKENS)
    lt_h, tbl = _tc_logits_t(hs,
                             W_proj.reshape(1, ENC_DIM),
                             b_proj.reshape(1, ENC_DIM),
                             gamma.reshape(1, ENC_DIM),
                             beta.reshape(1, ENC_DIM),
                             W_gate)
    lt, idx2, w2 = _sc_topk(lt_h, ea_flat, tbl)
    return (idx2.T, w2.T, lt.T)
